# baseline (speedup 1.0000x reference)
import functools

import jax
import jax.numpy as jnp
import numpy as np
from jax import lax
from jax.experimental import pallas as pl
from jax.experimental.pallas import tpu as pltpu
from jax.experimental.pallas import tpu_sc as plsc

N_NODES = 10000
N_EDGES = 320000
NFEAT = 128
NHID = 128
NCLASS = 40
FH = 64
CPAD = 48

K = 125
NCH1 = 160
NCH2 = 80
STRIPE = 640


def _make_sc_agg(F, nchunk, feat_split):
    mesh = plsc.VectorSubcoreMesh(core_axis_name="c", subcore_axis_name="s")
    gdtype = jnp.bfloat16 if feat_split else jnp.float32

    @functools.partial(
        pl.kernel,
        out_type=jax.ShapeDtypeStruct((2 * N_NODES, F), gdtype),
        mesh=mesh,
        compiler_params=pltpu.CompilerParams(
            needs_layout_passes=False, use_tc_tiling_on_sc=False
        ),
        scratch_types=[
            pltpu.VMEM((nchunk, K), jnp.int32),
            [pltpu.VMEM((K,), jnp.int32) for _ in range(4)],
            [pltpu.VMEM((K,), jnp.float32) for _ in range(4)],
            [pltpu.VMEM((K, F), gdtype) for _ in range(4)],
            pltpu.VMEM_SHARED((N_NODES, F), gdtype),
            [pltpu.SemaphoreType.DMA for _ in range(4)],
            [pltpu.SemaphoreType.DMA for _ in range(4)],
            [pltpu.SemaphoreType.DMA for _ in range(4)],
            [pltpu.SemaphoreType.DMA for _ in range(4)],
        ],
    )
    def sc_agg(h_hbm, src_hbm, tgtr_hbm, m_hbm, out_hbm,
               srcb, tgts, mbs, rowsb, accum, semg, semsc, semt, semm):
        c = lax.axis_index("c")
        s = lax.axis_index("s")
        w = s if feat_split else c * 16 + s

        def gather_cp(i, r):
            return pltpu.make_async_copy(
                h_hbm.at[srcb.at[i]], rowsb[r], semg[r]
            )

        def scatter_cp(b):
            return pltpu.make_async_copy(rowsb[b], accum.at[tgts[b]], semsc[b])

        def idx_cp(i, r):
            return (
                pltpu.make_async_copy(tgtr_hbm.at[w * nchunk + i], tgts[r], semt[r]),
                pltpu.make_async_copy(m_hbm.at[w * nchunk + i], mbs[r], semm[r]),
            )

        plane = c if feat_split else jnp.int32(0)
        pltpu.sync_copy(
            src_hbm.at[plane].at[pl.ds(w * nchunk, nchunk)], srcb
        )
        pltpu.sync_copy(tgtr_hbm.at[w * nchunk], tgts[0])
        pltpu.sync_copy(m_hbm.at[w * nchunk], mbs[0])

        rows0 = rowsb[0]

        def zrow(r, _):
            if feat_split:
                for g in range(F // 32):
                    rows0[r, pl.ds(g * 32, 32)] = jnp.zeros((32,), gdtype)
            else:
                for j in range(F // 16):
                    rows0[r, pl.ds(j * 16, 16)] = jnp.zeros((16,), jnp.float32)
            return 0
        lax.fori_loop(0, K, zrow, 0)
        for kk in range(STRIPE // 80):
            start = jnp.minimum(s * STRIPE + kk * 80, N_NODES - 80)
            pltpu.sync_copy(rows0.at[pl.ds(0, 80)], accum.at[pl.ds(start, 80)])
        plsc.subcore_barrier()

        lane0 = jnp.zeros((16,), jnp.int32)

        gather_cp(0, 0).start()
        for cp in idx_cp(1, 1):
            cp.start()

        def outer(o, _):
            for b in range(4):
                i = o * 4 + b
                r, rp, rn = b, (b + 3) % 4, (b + 1) % 4

                gather_cp(i, r).wait()

                @pl.when(i >= 1)
                def _():
                    scatter_cp(rp).wait()

                @pl.when(i + 1 < nchunk)
                def _():
                    for cp in idx_cp(i + 1, rn):
                        cp.wait()
                    gather_cp(i + 1, rn).start()

                if feat_split:
                    @plsc.parallel_loop(0, K, unroll=25)
                    def edge(e):
                        mv = plsc.load_gather(mbs[r], [lane0 + e])
                        mvb = plsc.pack(mv, mv, format=plsc.PackFormat.INTERLEAVED)
                        for g in range(F // 32):
                            sl = pl.ds(g * 32, 32)
                            rowsb[r][e, sl] = rowsb[r][e, sl] * mvb
                else:
                    @plsc.parallel_loop(0, K, unroll=25)
                    def edge(e):
                        mv = plsc.load_gather(mbs[r], [lane0 + e])
                        for j in range(F // 16):
                            sl = pl.ds(j * 16, 16)
                            rowsb[r][e, sl] = rowsb[r][e, sl] * mv

                scatter_cp(r).start(add=True)

                @pl.when(i + 2 < nchunk)
                def _():
                    for cp in idx_cp(i + 2, (b + 2) % 4):
                        cp.start()
            return 0
        lax.fori_loop(0, nchunk // 4, outer, 0)
        scatter_cp((nchunk - 1) % 4).wait()

        plsc.subcore_barrier()
        ostart = jnp.minimum(s * STRIPE, N_NODES - STRIPE)
        pltpu.sync_copy(
            accum.at[pl.ds(ostart, STRIPE)],
            out_hbm.at[pl.ds(c * N_NODES + ostart, STRIPE)],
        )

    return sc_agg


_sc_agg1 = _make_sc_agg(FH, NCH1, feat_split=True)
_sc_agg2 = _make_sc_agg(CPAD, NCH2, feat_split=False)

_BR = 1000
_G = N_NODES // _BR


def _mmf_body(p0_ref, p1_ref, w1_ref, b1_ref, w2_ref, o_ref):
    p0 = p0_ref[...].astype(jnp.float32)
    p1 = p1_ref[...].astype(jnp.float32)
    z = (
        jnp.dot(p0, w1_ref[0], preferred_element_type=jnp.float32)
        + jnp.dot(p1, w1_ref[1], preferred_element_type=jnp.float32)
        + b1_ref[...]
    )
    h = jnp.maximum(z, 0.0)
    o_ref[...] = jnp.dot(h, w2_ref[...], preferred_element_type=jnp.float32)


def _mmf(p, W1P, b1, W2p):
    return pl.pallas_call(
        _mmf_body,
        grid=(_G,),
        in_specs=[
            pl.BlockSpec((_BR, FH), lambda i: (i, 0)),
            pl.BlockSpec((_BR, FH), lambda i: (i + _G, 0)),
            pl.BlockSpec((2, FH, NHID), lambda i: (0, 0, 0)),
            pl.BlockSpec((1, NHID), lambda i: (0, 0)),
            pl.BlockSpec((NHID, CPAD), lambda i: (0, 0)),
        ],
        out_specs=pl.BlockSpec((_BR, CPAD), lambda i: (i, 0)),
        out_shape=jax.ShapeDtypeStruct((N_NODES, CPAD), jnp.float32),
    )(p, p, W1P, b1, W2p)


def _fin_body(q0_ref, q1_ref, b2_ref, o_ref):
    z = q0_ref[...] + q1_ref[...] + b2_ref[...]
    col = lax.broadcasted_iota(jnp.int32, z.shape, 1)
    zm = jnp.where(col < NCLASS, z, -jnp.inf)
    m = jnp.max(zm, axis=1, keepdims=True)
    ls = jnp.log(jnp.sum(jnp.exp(zm - m), axis=1, keepdims=True))
    o_ref[...] = (z - m - ls)[:, :NCLASS]


def _fin(q, b2p):
    return pl.pallas_call(
        _fin_body,
        grid=(_G,),
        in_specs=[
            pl.BlockSpec((_BR, CPAD), lambda i: (i, 0)),
            pl.BlockSpec((_BR, CPAD), lambda i: (i + _G, 0)),
            pl.BlockSpec((1, CPAD), lambda i: (0, 0)),
        ],
        out_specs=pl.BlockSpec((_BR, NCLASS), lambda i: (i, 0)),
        out_shape=jax.ShapeDtypeStruct((N_NODES, NCLASS), jnp.float32),
    )(q, q, b2p)


def kernel(x, src, tgt, Mtgt, W1, b1, W2, b2):
    src_i = src.astype(jnp.int32)
    src2 = jnp.stack([src_i, src_i + N_NODES]).reshape(2, N_EDGES // K, K)
    tgt2d = tgt.astype(jnp.int32).reshape(N_EDGES // K, K)
    m2d = Mtgt.reshape(N_EDGES // K, K)
    x2 = (
        x.reshape(N_NODES, 2, FH).transpose(1, 0, 2)
        .reshape(2 * N_NODES, FH).astype(jnp.bfloat16)
    )
    W1P = W1.reshape(2, FH, NHID)
    W2p = jnp.pad(W2, ((0, 0), (0, CPAD - NCLASS)))
    b2p = jnp.pad(b2, (0, CPAD - NCLASS)).reshape(1, CPAD)
    p = _sc_agg1(x2, src2, tgt2d, m2d)
    h2 = _mmf(p, W1P, b1.reshape(1, NHID), W2p)
    q = _sc_agg2(h2, src2, tgt2d, m2d)
    return _fin(q, b2p)


# speedup vs baseline: 1.0077x; 1.0077x over previous
import functools

import jax
import jax.numpy as jnp
import numpy as np
from jax import lax
from jax.experimental import pallas as pl
from jax.experimental.pallas import tpu as pltpu
from jax.experimental.pallas import tpu_sc as plsc

N_NODES = 10000
N_EDGES = 320000
NFEAT = 128
NHID = 128
NCLASS = 40
FH = 64
CPAD = 48

K = 125
NCH1 = 160
NCH2 = 80
STRIPE = 640


def _make_sc_agg(F, nchunk, feat_split):
    mesh = plsc.VectorSubcoreMesh(core_axis_name="c", subcore_axis_name="s")
    gdtype = jnp.bfloat16 if feat_split else jnp.float32

    @functools.partial(
        pl.kernel,
        out_type=jax.ShapeDtypeStruct((2 * N_NODES, F), gdtype),
        mesh=mesh,
        compiler_params=pltpu.CompilerParams(
            needs_layout_passes=False, use_tc_tiling_on_sc=False
        ),
        scratch_types=[
            pltpu.VMEM((nchunk, K), jnp.int32),
            [pltpu.VMEM((K,), jnp.int32) for _ in range(4)],
            [pltpu.VMEM((K,), jnp.float32) for _ in range(4)],
            [pltpu.VMEM((K, F), gdtype) for _ in range(4)],
            pltpu.VMEM_SHARED((N_NODES, F), gdtype),
            [pltpu.SemaphoreType.DMA for _ in range(4)],
            [pltpu.SemaphoreType.DMA for _ in range(4)],
            [pltpu.SemaphoreType.DMA for _ in range(4)],
            [pltpu.SemaphoreType.DMA for _ in range(4)],
        ],
    )
    def sc_agg(h_hbm, src_hbm, tgtr_hbm, m_hbm, out_hbm,
               srcb, tgts, mbs, rowsb, accum, semg, semsc, semt, semm):
        c = lax.axis_index("c")
        s = lax.axis_index("s")
        w = s if feat_split else c * 16 + s

        def gather_cp(i, r):
            return pltpu.make_async_copy(
                h_hbm.at[srcb.at[i]], rowsb[r], semg[r]
            )

        def scatter_cp(b):
            return pltpu.make_async_copy(rowsb[b], accum.at[tgts[b]], semsc[b])

        def idx_cp(i, r):
            return (
                pltpu.make_async_copy(tgtr_hbm.at[w * nchunk + i], tgts[r], semt[r]),
                pltpu.make_async_copy(m_hbm.at[w * nchunk + i], mbs[r], semm[r]),
            )

        plane = c if feat_split else jnp.int32(0)
        pltpu.sync_copy(
            src_hbm.at[plane].at[pl.ds(w * nchunk, nchunk)], srcb
        )
        pltpu.sync_copy(tgtr_hbm.at[w * nchunk], tgts[0])
        pltpu.sync_copy(m_hbm.at[w * nchunk], mbs[0])

        rows0 = rowsb[0]

        def zrow(r, _):
            if feat_split:
                for g in range(F // 32):
                    rows0[r, pl.ds(g * 32, 32)] = jnp.zeros((32,), gdtype)
            else:
                for j in range(F // 16):
                    rows0[r, pl.ds(j * 16, 16)] = jnp.zeros((16,), jnp.float32)
            return 0
        lax.fori_loop(0, K, zrow, 0)
        for kk in range(STRIPE // 80):
            start = jnp.minimum(s * STRIPE + kk * 80, N_NODES - 80)
            pltpu.sync_copy(rows0.at[pl.ds(0, 80)], accum.at[pl.ds(start, 80)])
        plsc.subcore_barrier()

        lane0 = jnp.zeros((16,), jnp.int32)

        gather_cp(0, 0).start()
        for cp in idx_cp(1, 1):
            cp.start()

        def outer(o, _):
            for b in range(4):
                i = o * 4 + b
                r, rp, rn = b, (b + 3) % 4, (b + 1) % 4

                gather_cp(i, r).wait()

                @pl.when(i >= 1)
                def _():
                    scatter_cp(rp).wait()

                @pl.when(i + 1 < nchunk)
                def _():
                    for cp in idx_cp(i + 1, rn):
                        cp.wait()
                    gather_cp(i + 1, rn).start()

                if feat_split:
                    @plsc.parallel_loop(0, K, unroll=5)
                    def edge(e):
                        mv = plsc.load_gather(mbs[r], [lane0 + e])
                        mvb = plsc.pack(mv, mv, format=plsc.PackFormat.INTERLEAVED)
                        for g in range(F // 32):
                            sl = pl.ds(g * 32, 32)
                            rowsb[r][e, sl] = rowsb[r][e, sl] * mvb
                else:
                    @plsc.parallel_loop(0, K, unroll=5)
                    def edge(e):
                        mv = plsc.load_gather(mbs[r], [lane0 + e])
                        for j in range(F // 16):
                            sl = pl.ds(j * 16, 16)
                            rowsb[r][e, sl] = rowsb[r][e, sl] * mv

                scatter_cp(r).start(add=True)

                @pl.when(i + 2 < nchunk)
                def _():
                    for cp in idx_cp(i + 2, (b + 2) % 4):
                        cp.start()
            return 0
        lax.fori_loop(0, nchunk // 4, outer, 0)
        scatter_cp((nchunk - 1) % 4).wait()

        plsc.subcore_barrier()
        ostart = jnp.minimum(s * STRIPE, N_NODES - STRIPE)
        pltpu.sync_copy(
            accum.at[pl.ds(ostart, STRIPE)],
            out_hbm.at[pl.ds(c * N_NODES + ostart, STRIPE)],
        )

    return sc_agg


_sc_agg1 = _make_sc_agg(FH, NCH1, feat_split=True)
_sc_agg2 = _make_sc_agg(CPAD, NCH2, feat_split=False)

_BR = 1000
_G = N_NODES // _BR


def _mmf_body(p0_ref, p1_ref, w1_ref, b1_ref, w2_ref, o_ref):
    p0 = p0_ref[...].astype(jnp.float32)
    p1 = p1_ref[...].astype(jnp.float32)
    z = (
        jnp.dot(p0, w1_ref[0], preferred_element_type=jnp.float32)
        + jnp.dot(p1, w1_ref[1], preferred_element_type=jnp.float32)
        + b1_ref[...]
    )
    h = jnp.maximum(z, 0.0)
    o_ref[...] = jnp.dot(h, w2_ref[...], preferred_element_type=jnp.float32)


def _mmf(p, W1P, b1, W2p):
    return pl.pallas_call(
        _mmf_body,
        grid=(_G,),
        in_specs=[
            pl.BlockSpec((_BR, FH), lambda i: (i, 0)),
            pl.BlockSpec((_BR, FH), lambda i: (i + _G, 0)),
            pl.BlockSpec((2, FH, NHID), lambda i: (0, 0, 0)),
            pl.BlockSpec((1, NHID), lambda i: (0, 0)),
            pl.BlockSpec((NHID, CPAD), lambda i: (0, 0)),
        ],
        out_specs=pl.BlockSpec((_BR, CPAD), lambda i: (i, 0)),
        out_shape=jax.ShapeDtypeStruct((N_NODES, CPAD), jnp.float32),
    )(p, p, W1P, b1, W2p)


def _fin_body(q0_ref, q1_ref, b2_ref, o_ref):
    z = q0_ref[...] + q1_ref[...] + b2_ref[...]
    col = lax.broadcasted_iota(jnp.int32, z.shape, 1)
    zm = jnp.where(col < NCLASS, z, -jnp.inf)
    m = jnp.max(zm, axis=1, keepdims=True)
    ls = jnp.log(jnp.sum(jnp.exp(zm - m), axis=1, keepdims=True))
    o_ref[...] = (z - m - ls)[:, :NCLASS]


def _fin(q, b2p):
    return pl.pallas_call(
        _fin_body,
        grid=(_G,),
        in_specs=[
            pl.BlockSpec((_BR, CPAD), lambda i: (i, 0)),
            pl.BlockSpec((_BR, CPAD), lambda i: (i + _G, 0)),
            pl.BlockSpec((1, CPAD), lambda i: (0, 0)),
        ],
        out_specs=pl.BlockSpec((_BR, NCLASS), lambda i: (i, 0)),
        out_shape=jax.ShapeDtypeStruct((N_NODES, NCLASS), jnp.float32),
    )(q, q, b2p)


def kernel(x, src, tgt, Mtgt, W1, b1, W2, b2):
    src_i = src.astype(jnp.int32)
    src2 = jnp.stack([src_i, src_i + N_NODES]).reshape(2, N_EDGES // K, K)
    tgt2d = tgt.astype(jnp.int32).reshape(N_EDGES // K, K)
    m2d = Mtgt.reshape(N_EDGES // K, K)
    x2 = (
        x.reshape(N_NODES, 2, FH).transpose(1, 0, 2)
        .reshape(2 * N_NODES, FH).astype(jnp.bfloat16)
    )
    W1P = W1.reshape(2, FH, NHID)
    W2p = jnp.pad(W2, ((0, 0), (0, CPAD - NCLASS)))
    b2p = jnp.pad(b2, (0, CPAD - NCLASS)).reshape(1, CPAD)
    p = _sc_agg1(x2, src2, tgt2d, m2d)
    h2 = _mmf(p, W1P, b1.reshape(1, NHID), W2p)
    q = _sc_agg2(h2, src2, tgt2d, m2d)
    return _fin(q, b2p)
